# Initial kernel scaffold; baseline (speedup 1.0000x reference)
import jax
import jax.numpy as jnp
from jax.experimental import pallas as pl


def kernel(x, emb, W1, b1, W2, b2, W3, b3):
    raise NotImplementedError("write your pallas kernel here")


# speedup vs baseline: 23.5390x; 23.5390x over previous
import functools

import jax
import jax.numpy as jnp
from jax import lax
from jax.experimental import pallas as pl
from jax.experimental.pallas import tpu as pltpu
from jax.experimental.pallas import tpu_sc as plsc

B, T, V, E = 4096, 200, 1000000, 32
H1, H2, H3 = 500, 100, 1
N = B * T

_info = plsc.get_sparse_core_info()
NC, NS, L = _info.num_cores, _info.num_subcores, _info.num_lanes
NW = NC * NS

B_PER_W = N // NW
CHUNK = 128
N_CHUNKS = B_PER_W // CHUNK

_mesh = plsc.VectorSubcoreMesh(core_axis_name="c", subcore_axis_name="s")


@functools.partial(
    pl.kernel,
    mesh=_mesh,
    out_type=jax.ShapeDtypeStruct((N, E), jnp.float32),
    scratch_types=[
        pltpu.VMEM((N_CHUNKS, CHUNK), jnp.int32),
        pltpu.VMEM((2, CHUNK, E), jnp.float32),
        pltpu.SemaphoreType.DMA,
        pltpu.SemaphoreType.DMA,
    ],
    compiler_params=pltpu.CompilerParams(use_tc_tiling_on_sc=False),
)
def _gather_sc(idx_hbm, table_hbm, out_hbm, idx_v, rows_v, gsem, wsem):
    wid = lax.axis_index("s") * NC + lax.axis_index("c")
    base = wid * B_PER_W
    pltpu.sync_copy(idx_hbm.at[wid], idx_v)

    def body(j, _):
        buf = lax.rem(j, 2)
        g = pltpu.make_async_copy(table_hbm.at[idx_v.at[j]], rows_v.at[buf], gsem)
        g.start()
        g.wait()
        w = pltpu.make_async_copy(
            rows_v.at[buf], out_hbm.at[pl.ds(base + j * CHUNK, CHUNK)], wsem
        )
        w.start()
        w.wait()
        return ()

    lax.fori_loop(0, N_CHUNKS, body, ())


def _mlp_body(h_ref, w1_ref, b1_ref, w2_ref, b2_ref, w3_ref, b3_ref, o_ref):
    h = h_ref[...]
    a1 = jnp.dot(h, w1_ref[...], preferred_element_type=jnp.float32)
    a1 = jnp.maximum(a1 + b1_ref[...], 0.0)
    a2 = jnp.dot(a1, w2_ref[...], preferred_element_type=jnp.float32)
    a2 = jnp.maximum(a2 + b2_ref[...], 0.0)
    a3 = jnp.dot(a2, w3_ref[...], preferred_element_type=jnp.float32)
    o_ref[...] = jax.nn.sigmoid(a3 + b3_ref[...])


BM = 512


def _mlp_tc(h, W1, b1, W2, b2, W3, b3):
    grid = (B // BM,)
    return pl.pallas_call(
        _mlp_body,
        grid=grid,
        in_specs=[
            pl.BlockSpec((BM, T * E), lambda i: (i, 0)),
            pl.BlockSpec((T * E, H1), lambda i: (0, 0)),
            pl.BlockSpec((1, H1), lambda i: (0, 0)),
            pl.BlockSpec((H1, H2), lambda i: (0, 0)),
            pl.BlockSpec((1, H2), lambda i: (0, 0)),
            pl.BlockSpec((H2, H3), lambda i: (0, 0)),
            pl.BlockSpec((1, H3), lambda i: (0, 0)),
        ],
        out_specs=pl.BlockSpec((BM, H3), lambda i: (i, 0)),
        out_shape=jax.ShapeDtypeStruct((B, H3), jnp.float32),
    )(h, W1, b1.reshape(1, H1), W2, b2.reshape(1, H2), W3, b3.reshape(1, H3))


@jax.jit
def kernel(x, emb, W1, b1, W2, b2, W3, b3):
    x_flat = x.reshape(NW, N_CHUNKS, CHUNK)
    rows = _gather_sc(x_flat, emb)
    h = rows.reshape(B, T * E)
    return _mlp_tc(h, W1, b1, W2, b2, W3, b3)


# speedup vs baseline: 25.2754x; 1.0738x over previous
import functools

import jax
import jax.numpy as jnp
from jax import lax
from jax.experimental import pallas as pl
from jax.experimental.pallas import tpu as pltpu
from jax.experimental.pallas import tpu_sc as plsc

B, T, V, E = 4096, 200, 1000000, 32
H1, H2, H3 = 500, 100, 1
N = B * T
NCH = (T * E) // 128

NC, NS = 2, 16
NW = NC * NS

B_PER_W = N // NW
CHUNK = 128
N_CHUNKS = B_PER_W // CHUNK


def _gather_sc_body(idx_hbm, dst_hbm, table_hbm, out_hbm,
                    idx_v, dst_v, rows_v, gsem, wsem):
    wid = lax.axis_index("s") * NC + lax.axis_index("c")
    pltpu.sync_copy(idx_hbm.at[wid], idx_v)
    pltpu.sync_copy(dst_hbm.at[wid], dst_v)

    def body(j, _):
        buf = lax.rem(j, 2)
        g = pltpu.make_async_copy(table_hbm.at[idx_v.at[j]], rows_v.at[buf], gsem)
        g.start()
        g.wait()
        w = pltpu.make_async_copy(rows_v.at[buf], out_hbm.at[dst_v.at[j]], wsem)
        w.start()
        w.wait()
        return ()

    lax.fori_loop(0, N_CHUNKS, body, ())


@functools.cache
def _gather_sc():
    mesh = plsc.VectorSubcoreMesh(
        core_axis_name="c", subcore_axis_name="s", num_cores=NC
    )
    return pl.kernel(
        _gather_sc_body,
        mesh=mesh,
        out_type=jax.ShapeDtypeStruct((N, E), jnp.float32),
        scratch_types=[
            pltpu.VMEM((N_CHUNKS, CHUNK), jnp.int32),
            pltpu.VMEM((N_CHUNKS, CHUNK), jnp.int32),
            pltpu.VMEM((2, CHUNK, E), jnp.float32),
            pltpu.SemaphoreType.DMA,
            pltpu.SemaphoreType.DMA,
        ],
        compiler_params=pltpu.CompilerParams(use_tc_tiling_on_sc=False),
    )


def _dst_map():
    m = jnp.arange(N, dtype=jnp.int32)
    b = m // T
    t = m - b * T
    p = (t // 4 * B + b) * 4 + (t - t // 4 * 4)
    return p.reshape(NW, N_CHUNKS, CHUNK)


def _mlp_body(h_ref, w1t_ref, b1_ref, w2t_ref, b2_ref, w3_ref, b3_ref,
              o_ref, acc_ref):
    c = pl.program_id(0)

    @pl.when(c == 0)
    def _init():
        acc_ref[...] = jnp.zeros_like(acc_ref)

    acc_ref[...] += lax.dot_general(
        h_ref[0], w1t_ref[...], (((1,), (1,)), ((), ())),
        preferred_element_type=jnp.float32,
    )

    @pl.when(c == NCH - 1)
    def _finish():
        a1 = jnp.maximum(acc_ref[...] + b1_ref[...], 0.0)
        a2 = lax.dot_general(
            a1, w2t_ref[...], (((1,), (1,)), ((), ())),
            preferred_element_type=jnp.float32,
        )
        a2 = jnp.maximum(a2 + b2_ref[...], 0.0)
        a3 = jnp.dot(a2, w3_ref[...], preferred_element_type=jnp.float32)
        o_ref[...] = jax.nn.sigmoid(a3 + b3_ref[...])


def _mlp_tc(h2, W1t, b1, W2t, b2, W3, b3):
    return pl.pallas_call(
        _mlp_body,
        grid=(NCH,),
        in_specs=[
            pl.BlockSpec((1, B, 128), lambda c: (c, 0, 0)),
            pl.BlockSpec((H1, 128), lambda c: (0, c)),
            pl.BlockSpec((1, H1), lambda c: (0, 0)),
            pl.BlockSpec((H2, H1), lambda c: (0, 0)),
            pl.BlockSpec((1, H2), lambda c: (0, 0)),
            pl.BlockSpec((H2, H3), lambda c: (0, 0)),
            pl.BlockSpec((1, H3), lambda c: (0, 0)),
        ],
        out_specs=pl.BlockSpec((B, H3), lambda c: (0, 0)),
        out_shape=jax.ShapeDtypeStruct((B, H3), jnp.float32),
        scratch_shapes=[pltpu.VMEM((B, H1), jnp.float32)],
    )(h2, W1t, b1.reshape(1, H1), W2t, b2.reshape(1, H2), W3,
      b3.reshape(1, H3))


@jax.jit
def kernel(x, emb, W1, b1, W2, b2, W3, b3):
    xm = x.reshape(NW, N_CHUNKS, CHUNK)
    rows = _gather_sc()(xm, _dst_map(), emb)
    h2 = rows.reshape(NCH, B, 128)
    return _mlp_tc(h2, W1.T, b1, W2.T, b2, W3, b3)
